# baseline (speedup 1.0000x reference)
import jax
import jax.numpy as jnp
from jax import lax
from jax.experimental import pallas as pl
from jax.experimental.pallas import tpu as pltpu
from jax.experimental.pallas import tpu_sc as plsc

NC = 2
NS = 16
NW = NC * NS
DEGW = 128
K = 128
BPW = 80


def _make_sc_agg(n_pad, d):
    mesh = plsc.VectorSubcoreMesh(core_axis_name="c", subcore_axis_name="s")

    out_type = jax.ShapeDtypeStruct((NC * n_pad, d), jnp.float32)
    scratch = [
        pltpu.VMEM((BPW, K), jnp.int32),
        pltpu.VMEM((BPW, K), jnp.int32),
        pltpu.VMEM((K, d), jnp.float32),
        pltpu.VMEM_SHARED((n_pad, d), jnp.float32),
        pltpu.SemaphoreType.DMA,
    ]

    def body(x_hbm, src_hbm, dst_hbm, z_hbm, agg_out,
             src_all, dst_all, rows_v, agg_s, sem):
        ci = lax.axis_index("c")
        si = lax.axis_index("s")
        wid = ci * NS + si

        @pl.when(si == 0)
        def _init():
            pltpu.sync_copy(z_hbm, agg_s)

        pltpu.sync_copy(src_hbm.at[pl.ds(wid * BPW, BPW), :], src_all)
        pltpu.sync_copy(dst_hbm.at[pl.ds(wid * BPW, BPW), :], dst_all)

        plsc.subcore_barrier()

        @pl.loop(0, BPW)
        def _blk(i):
            pltpu.async_copy(x_hbm.at[src_all.at[i]], rows_v, sem).wait()
            pltpu.sync_copy(rows_v, agg_s.at[dst_all.at[i]], add=True)

        plsc.subcore_barrier()

        @pl.when(si == 0)
        def _drain():
            pltpu.sync_copy(agg_s, agg_out.at[pl.ds(ci * n_pad, n_pad), :])

    return pl.kernel(body, out_type=out_type, mesh=mesh, scratch_types=scratch)


def _make_sc_deg(n_pad):
    mesh = plsc.VectorSubcoreMesh(core_axis_name="c", subcore_axis_name="s")

    out_type = jax.ShapeDtypeStruct((NC * n_pad, DEGW), jnp.float32)
    scratch = [
        pltpu.VMEM((BPW, K), jnp.int32),
        pltpu.VMEM((K, DEGW), jnp.float32),
        pltpu.VMEM_SHARED((n_pad, DEGW), jnp.float32),
    ]

    def body(dst_hbm, zd_hbm, ones_hbm, deg_out, dst_all, ones_v, deg_s):
        ci = lax.axis_index("c")
        si = lax.axis_index("s")
        wid = ci * NS + si

        @pl.when(si == 0)
        def _init():
            pltpu.sync_copy(zd_hbm, deg_s)

        pltpu.sync_copy(ones_hbm, ones_v)
        pltpu.sync_copy(dst_hbm.at[pl.ds(wid * BPW, BPW), :], dst_all)

        plsc.subcore_barrier()

        @pl.loop(0, BPW)
        def _blk(i):
            pltpu.sync_copy(ones_v, deg_s.at[dst_all.at[i]], add=True)

        plsc.subcore_barrier()

        @pl.when(si == 0)
        def _drain():
            pltpu.sync_copy(deg_s, deg_out.at[pl.ds(ci * n_pad, n_pad), :])

    return pl.kernel(body, out_type=out_type, mesh=mesh, scratch_types=scratch)


def _tc_layer1(agg, deg, x, Wl, Wr, b):
    n, d = x.shape
    n_pad = agg.shape[0] // NC

    def body(agg_ref, deg_ref, x_ref, wl_ref, wr_ref, b_ref, h_ref):
        s = agg_ref[:n, :] + agg_ref[n_pad:n_pad + n, :]
        dv = deg_ref[:n, :] + deg_ref[n_pad:n_pad + n, :]
        mean = s / jnp.maximum(dv[:, :1], 1.0)
        acc = jnp.dot(mean, wl_ref[...], preferred_element_type=jnp.float32)
        acc = acc + jnp.dot(x_ref[...], wr_ref[...],
                            preferred_element_type=jnp.float32)
        acc = acc + b_ref[...]
        h_ref[...] = jnp.maximum(acc, 0.0)

    return pl.pallas_call(
        body, out_shape=jax.ShapeDtypeStruct((n, d), jnp.float32),
    )(agg, deg, x, Wl, Wr, b.reshape(1, d))


def _tc_layer2(agg, deg, h, Wl, Wr, b):
    n, d = h.shape
    n_pad = agg.shape[0] // NC

    def body(agg_ref, deg_ref, h_ref, wl_ref, wr_ref, b_ref, o_ref, ls_ref):
        s = agg_ref[:n, :] + agg_ref[n_pad:n_pad + n, :]
        dv = deg_ref[:n, :] + deg_ref[n_pad:n_pad + n, :]
        mean = s / jnp.maximum(dv[:, :1], 1.0)
        o = jnp.dot(mean, wl_ref[...], preferred_element_type=jnp.float32)
        o = o + jnp.dot(h_ref[...], wr_ref[...],
                        preferred_element_type=jnp.float32)
        o = o + b_ref[...]
        o_ref[...] = o
        m = jnp.max(o, axis=1, keepdims=True)
        lse = jnp.log(jnp.sum(jnp.exp(o - m), axis=1, keepdims=True)) + m
        ls_ref[...] = o - lse

    return pl.pallas_call(
        body,
        out_shape=(jax.ShapeDtypeStruct((n, d), jnp.float32),
                   jax.ShapeDtypeStruct((n, d), jnp.float32)),
    )(agg, deg, h, Wl, Wr, b.reshape(1, d))


def kernel(x, edge_index, W1l, W1r, b1, W2l, W2r, b2):
    n, d = x.shape
    e = edge_index.shape[1]
    ei = edge_index.astype(jnp.int32)
    src, dst = ei[0], ei[1]

    n_pad = -(-n // (NS * 8)) * (NS * 8)
    zeros = jnp.zeros((n_pad, d), jnp.float32)
    zeros_deg = jnp.zeros((n_pad, DEGW), jnp.float32)
    ones = jnp.ones((K, DEGW), jnp.float32)

    e_pad = NW * BPW * K
    pad = e_pad - e
    src_p = jnp.concatenate([src, jnp.zeros((pad,), jnp.int32)])
    dst_p = jnp.concatenate(
        [dst, n + (jnp.arange(pad, dtype=jnp.int32) % (n_pad - n))])
    src2 = src_p.reshape(NW * BPW, K)
    dst2 = dst_p.reshape(NW * BPW, K)

    deg = _make_sc_deg(n_pad)(dst2, zeros_deg, ones)
    agg1 = _make_sc_agg(n_pad, d)(x, src2, dst2, zeros)
    h = _tc_layer1(agg1, deg, x, W1l, W1r, b1)
    agg2 = _make_sc_agg(n_pad, d)(h, src2, dst2, zeros)
    out, ls = _tc_layer2(agg2, deg, h, W2l, W2r, b2)
    return (out, ls)


# speedup vs baseline: 1.6067x; 1.6067x over previous
import jax
import jax.numpy as jnp
from jax import lax
from jax.experimental import pallas as pl
from jax.experimental.pallas import tpu as pltpu
from jax.experimental.pallas import tpu_sc as plsc

NC = 2
NS = 16
NW = NC * NS
DEGW = 128
K = 128
BPW = 80
KA = 80


def _make_sc_agg(n_pad, d, n_edges):
    mesh = plsc.VectorSubcoreMesh(core_axis_name="c", subcore_axis_name="s")
    epw = n_edges // NW
    nblk = epw // KA

    out_type = jax.ShapeDtypeStruct((NC * n_pad, d), jnp.float32)
    scratch = [
        pltpu.VMEM((KA,), jnp.int32),
        pltpu.VMEM((KA,), jnp.int32),
        pltpu.VMEM((KA, d), jnp.float32),
        pltpu.VMEM_SHARED((n_pad, d), jnp.float32),
        pltpu.SemaphoreType.DMA,
    ]

    def body(x_hbm, src_hbm, dst_hbm, z_hbm, agg_out,
             src_v, dst_v, rows_v, agg_s, sem):
        ci = lax.axis_index("c")
        si = lax.axis_index("s")
        wid = ci * NS + si

        @pl.when(si == 0)
        def _init():
            pltpu.sync_copy(z_hbm, agg_s)

        plsc.subcore_barrier()

        base = wid * epw

        @pl.loop(0, nblk)
        def _blk(i):
            off = base + i * KA
            pltpu.sync_copy(src_hbm.at[pl.ds(off, KA)], src_v)
            pltpu.sync_copy(dst_hbm.at[pl.ds(off, KA)], dst_v)
            pltpu.async_copy(x_hbm.at[src_v], rows_v, sem).wait()
            pltpu.sync_copy(rows_v, agg_s.at[dst_v], add=True)

        plsc.subcore_barrier()

        @pl.when(si == 0)
        def _drain():
            pltpu.sync_copy(agg_s, agg_out.at[pl.ds(ci * n_pad, n_pad), :])

    return pl.kernel(body, out_type=out_type, mesh=mesh, scratch_types=scratch)


def _make_sc_deg(n_pad):
    mesh = plsc.VectorSubcoreMesh(core_axis_name="c", subcore_axis_name="s")

    out_type = jax.ShapeDtypeStruct((NC * n_pad, DEGW), jnp.float32)
    scratch = [
        pltpu.VMEM((BPW, K), jnp.int32),
        pltpu.VMEM((K, DEGW), jnp.float32),
        pltpu.VMEM_SHARED((n_pad, DEGW), jnp.float32),
    ]

    def body(dst_hbm, zd_hbm, ones_hbm, deg_out, dst_all, ones_v, deg_s):
        ci = lax.axis_index("c")
        si = lax.axis_index("s")
        wid = ci * NS + si

        @pl.when(si == 0)
        def _init():
            pltpu.sync_copy(zd_hbm, deg_s)

        pltpu.sync_copy(ones_hbm, ones_v)
        pltpu.sync_copy(dst_hbm.at[pl.ds(wid * BPW, BPW), :], dst_all)

        plsc.subcore_barrier()

        @pl.loop(0, BPW)
        def _blk(i):
            pltpu.sync_copy(ones_v, deg_s.at[dst_all.at[i]], add=True)

        plsc.subcore_barrier()

        @pl.when(si == 0)
        def _drain():
            pltpu.sync_copy(deg_s, deg_out.at[pl.ds(ci * n_pad, n_pad), :])

    return pl.kernel(body, out_type=out_type, mesh=mesh, scratch_types=scratch)


def _tc_layer1(agg, deg, x, Wl, Wr, b):
    n, d = x.shape
    n_pad = agg.shape[0] // NC

    def body(agg_ref, deg_ref, x_ref, wl_ref, wr_ref, b_ref, h_ref):
        s = agg_ref[:n, :] + agg_ref[n_pad:n_pad + n, :]
        dv = deg_ref[:n, :] + deg_ref[n_pad:n_pad + n, :]
        mean = s / jnp.maximum(dv[:, :1], 1.0)
        acc = jnp.dot(mean, wl_ref[...], preferred_element_type=jnp.float32)
        acc = acc + jnp.dot(x_ref[...], wr_ref[...],
                            preferred_element_type=jnp.float32)
        acc = acc + b_ref[...]
        h_ref[...] = jnp.maximum(acc, 0.0)

    return pl.pallas_call(
        body, out_shape=jax.ShapeDtypeStruct((n, d), jnp.float32),
    )(agg, deg, x, Wl, Wr, b.reshape(1, d))


def _tc_layer2(agg, deg, h, Wl, Wr, b):
    n, d = h.shape
    n_pad = agg.shape[0] // NC

    def body(agg_ref, deg_ref, h_ref, wl_ref, wr_ref, b_ref, o_ref, ls_ref):
        s = agg_ref[:n, :] + agg_ref[n_pad:n_pad + n, :]
        dv = deg_ref[:n, :] + deg_ref[n_pad:n_pad + n, :]
        mean = s / jnp.maximum(dv[:, :1], 1.0)
        o = jnp.dot(mean, wl_ref[...], preferred_element_type=jnp.float32)
        o = o + jnp.dot(h_ref[...], wr_ref[...],
                        preferred_element_type=jnp.float32)
        o = o + b_ref[...]
        o_ref[...] = o
        m = jnp.max(o, axis=1, keepdims=True)
        lse = jnp.log(jnp.sum(jnp.exp(o - m), axis=1, keepdims=True)) + m
        ls_ref[...] = o - lse

    return pl.pallas_call(
        body,
        out_shape=(jax.ShapeDtypeStruct((n, d), jnp.float32),
                   jax.ShapeDtypeStruct((n, d), jnp.float32)),
    )(agg, deg, h, Wl, Wr, b.reshape(1, d))


def kernel(x, edge_index, W1l, W1r, b1, W2l, W2r, b2):
    n, d = x.shape
    e = edge_index.shape[1]
    ei = edge_index.astype(jnp.int32)
    src, dst = ei[0], ei[1]

    n_pad = -(-n // (NS * 8)) * (NS * 8)
    zeros = jnp.zeros((n_pad, d), jnp.float32)
    zeros_deg = jnp.zeros((n_pad, DEGW), jnp.float32)
    ones = jnp.ones((K, DEGW), jnp.float32)

    e_pad = NW * BPW * K
    pad = e_pad - e
    src_p = jnp.concatenate([src, jnp.zeros((pad,), jnp.int32)])
    dst_p = jnp.concatenate(
        [dst, n + (jnp.arange(pad, dtype=jnp.int32) % (n_pad - n))])
    src2 = src_p.reshape(NW * BPW, K)
    dst2 = dst_p.reshape(NW * BPW, K)

    deg = _make_sc_deg(n_pad)(dst2, zeros_deg, ones)
    agg1 = _make_sc_agg(n_pad, d, e)(x, src, dst, zeros)
    h = _tc_layer1(agg1, deg, x, W1l, W1r, b1)
    agg2 = _make_sc_agg(n_pad, d, e)(h, src, dst, zeros)
    out, ls = _tc_layer2(agg2, deg, h, W2l, W2r, b2)
    return (out, ls)
